# baseline (speedup 1.0000x reference)
import functools

import jax
import jax.numpy as jnp
from jax import lax
from jax.experimental import pallas as pl
from jax.experimental.pallas import tpu as pltpu

E_CNT = 10000
R_CNT = 1000
D = 300
S = 2048
T = 32768
NE = 160000
NH = 8
DK = 32
DV = 32


def _loss_body(lem_ref, rem_ref, out_ref, acc_ref):
    i = pl.program_id(0)
    nb = pl.num_programs(0)
    blk = lem_ref.shape[0]
    a = lem_ref[...]
    b = rem_ref[...]
    an = a / (jnp.sqrt(jnp.sum(a * a, axis=1, keepdims=True)) + 1e-9)
    bn = b / (jnp.sqrt(jnp.sum(b * b, axis=1, keepdims=True)) + 1e-9)
    sim = jax.lax.dot_general(an, bn, (((1,), (1,)), ((), ())),
                              preferred_element_type=jnp.float32)
    rows = jax.lax.broadcasted_iota(jnp.int32, sim.shape, 0) + i * blk
    cols = jax.lax.broadcasted_iota(jnp.int32, sim.shape, 1)
    diag_mask = rows == cols
    d = jnp.sum(jnp.where(diag_mask, sim, 0.0), axis=1)
    pos_c = jnp.sum(jnp.minimum(d, 0.9))
    neg_mat = jnp.where(diag_mask, 0.2, jnp.maximum(sim, 0.2))
    neg_c = jnp.sum(neg_mat)

    @pl.when(i == 0)
    def _():
        acc_ref[0, 0] = 0.0
        acc_ref[0, 1] = 0.0

    acc_ref[0, 0] += pos_c
    acc_ref[0, 1] += neg_c

    @pl.when(i == nb - 1)
    def _():
        n = jnp.float32(S)
        pos_part = acc_ref[0, 0] / n
        neg_part = acc_ref[0, 1] / (n * n)
        out_ref[0] = neg_part - 0.2 - pos_part + 0.9


def _loss_pallas(lem, rem):
    blk = 256
    grid = (S // blk,)
    return pl.pallas_call(
        _loss_body,
        grid=grid,
        in_specs=[
            pl.BlockSpec((blk, lem.shape[1]), lambda i: (i, 0)),
            pl.BlockSpec((S, rem.shape[1]), lambda i: (0, 0)),
        ],
        out_specs=pl.BlockSpec((1,), lambda i: (0,), memory_space=pltpu.SMEM),
        out_shape=jax.ShapeDtypeStruct((1,), jnp.float32),
        scratch_shapes=[pltpu.SMEM((1, 2), jnp.float32)],
    )(lem, rem)[0]


def _ragged_attn(rel_seq, head_seq, seg_ids, kernel_em, ent_feat, rel_feat,
                 Wq, Wk, Wv, Wo, n_seg):
    tok = rel_feat[rel_seq] + ent_feat[head_seq]
    q = (kernel_em @ Wq).reshape(n_seg, NH, DK)
    k = (tok @ Wk).reshape(-1, NH, DK)
    v = (tok @ Wv).reshape(-1, NH, DV)
    logits = jnp.einsum('thd,thd->th', k, q[seg_ids]) / jnp.sqrt(DK)
    m = jax.ops.segment_max(logits, seg_ids, num_segments=n_seg)
    m = jax.lax.stop_gradient(jnp.where(jnp.isfinite(m), m, 0.0))
    e = jnp.exp(logits - m[seg_ids])
    z = jax.ops.segment_sum(e, seg_ids, num_segments=n_seg)
    attn = e / (z[seg_ids] + 1e-9)
    pooled = jax.ops.segment_sum(attn[:, :, None] * v, seg_ids,
                                 num_segments=n_seg)
    return pooled.reshape(n_seg, NH * DV) @ Wo


def kernel(e_x, r_x, prim_adj, rela_adj, train_len, l_ent_ids, r_ent_ids,
           li_rel_seq, li_head_seq, li_seg_ids, lo_rel_seq, lo_head_seq,
           lo_seg_ids, ri_rel_seq, ri_head_seq, ri_seg_ids, ro_rel_seq,
           ro_head_seq, ro_seg_ids, Wrel, brel, W1, b1, W2, b2, Wq_i, Wk_i,
           Wv_i, Wo_i, Wq_o, Wk_o, Wv_o, Wo_o, Wprox, bprox):
    rel0 = r_x @ Wrel + brel
    X = jnp.concatenate([e_x, rel0], axis=0)
    N = E_CNT + R_CNT
    agg1 = jax.ops.segment_sum(X[prim_adj[0]], prim_adj[1], num_segments=N)
    H1 = jax.nn.relu(agg1 @ W1 + b1)
    agg2 = jax.ops.segment_sum(H1[rela_adj[0]], rela_adj[1], num_segments=N)
    out = agg2 @ W2 + b2
    gcn_ex = out[:E_CNT]
    gcn_rx = out[E_CNT:]
    Sn = l_ent_ids.shape[0]

    def prox(ent_ids, irel, ihead, iseg, orel, ohead, oseg):
        kernel_em = gcn_ex[ent_ids]
        in_em = _ragged_attn(irel, ihead, iseg, kernel_em, gcn_ex, gcn_rx,
                             Wq_i, Wk_i, Wv_i, Wo_i, Sn)
        out_em = _ragged_attn(orel, ohead, oseg, kernel_em, gcn_ex, gcn_rx,
                              Wq_o, Wk_o, Wv_o, Wo_o, Sn)
        p = jnp.concatenate([in_em, out_em], axis=1) @ Wprox + bprox
        return jnp.concatenate([kernel_em, p], axis=1)

    lem = prox(l_ent_ids, li_rel_seq, li_head_seq, li_seg_ids,
               lo_rel_seq, lo_head_seq, lo_seg_ids)
    rem = prox(r_ent_ids, ri_rel_seq, ri_head_seq, ri_seg_ids,
               ro_rel_seq, ro_head_seq, ro_seg_ids)
    loss = _loss_pallas(lem, rem)
    return (loss, lem, rem)


# speedup vs baseline: 1.0446x; 1.0446x over previous
import functools

import jax
import jax.numpy as jnp
from jax import lax
from jax.experimental import pallas as pl
from jax.experimental.pallas import tpu as pltpu
from jax.experimental.pallas import tpu_sc as plsc

E_CNT = 10000
R_CNT = 1000
D = 300
S = 2048
T = 32768
NE = 160000
NH = 8
DK = 32
DV = 32


N_NODES = E_CNT + R_CNT
SC_NC = 2
SC_NS = 16
FCOLS = 80
NSLICE = 4
ROWS = 11264
RPT = ROWS // SC_NS
ECH = 128
NEP = 163840
EPT = NEP // SC_NS
NCHUNK = EPT // ECH


def _sc_segsum_body(tabs, src2, dst2, out_h, sidx, didx, rows, acc, sem):
    c = lax.axis_index("c")
    t = lax.axis_index("s")

    for r in range(NSLICE // SC_NC):
        sl = c + SC_NC * r
        row_off = sl * N_NODES

        def zb(i, carry):
            for j in range(FCOLS // 16):
                rows[i, pl.ds(j * 16, 16)] = jnp.zeros((16,), jnp.float32)
            return carry

        lax.fori_loop(0, ECH, zb, 0)
        for z in range((RPT + ECH - 1) // ECH):
            nz = min(ECH, RPT - z * ECH)
            pltpu.sync_copy(rows.at[pl.ds(0, nz)],
                            acc.at[pl.ds(t * RPT + z * ECH, nz)])
        plsc.subcore_barrier()

        def step(j, carry):
            chunk = t * NCHUNK + j
            pltpu.sync_copy(src2.at[chunk], sidx)
            pltpu.sync_copy(dst2.at[chunk], didx.at[0])
            for i in range(ECH // 16):
                sidx[pl.ds(i * 16, 16)] = sidx[pl.ds(i * 16, 16)] + row_off
            pltpu.async_copy(tabs.at[sidx], rows, sem).wait()
            pltpu.sync_copy(rows, acc.at[didx.at[0]], add=True)
            return carry

        lax.fori_loop(0, NCHUNK, step, 0)
        plsc.subcore_barrier()
        pltpu.sync_copy(acc.at[pl.ds(t * RPT, RPT)],
                        out_h.at[pl.ds(sl * ROWS + t * RPT, RPT)])
        plsc.subcore_barrier()


def _sc_segsum(tabs, src2, dst2):
    mesh = plsc.VectorSubcoreMesh(core_axis_name="c", subcore_axis_name="s")
    f = pl.kernel(
        _sc_segsum_body,
        out_type=jax.ShapeDtypeStruct((NSLICE * ROWS, FCOLS), jnp.float32),
        mesh=mesh,
        scratch_types=[
            pltpu.VMEM((ECH,), jnp.int32),
            pltpu.VMEM((1, ECH), jnp.int32),
            pltpu.VMEM((ECH, FCOLS), jnp.float32),
            pltpu.VMEM_SHARED((ROWS, FCOLS), jnp.float32),
            pltpu.SemaphoreType.DMA,
        ],
        compiler_params=pltpu.CompilerParams(use_tc_tiling_on_sc=False),
    )
    return f(tabs, src2, dst2)


def _gcn_segment_sum(X, src, dst):
    Xp = jnp.pad(X, ((0, 0), (0, NSLICE * FCOLS - D)))
    tabs = jnp.concatenate(
        [Xp[:, s * FCOLS:(s + 1) * FCOLS] for s in range(NSLICE)], axis=0)
    pad = NEP - NE
    src2 = jnp.concatenate([src, jnp.zeros((pad,), src.dtype)]).reshape(-1, ECH)
    dst2 = jnp.concatenate(
        [dst, jnp.full((pad,), N_NODES, dst.dtype)]).reshape(-1, ECH)
    out = _sc_segsum(tabs, src2, dst2)
    return jnp.concatenate(
        [out[s * ROWS:s * ROWS + N_NODES] for s in range(NSLICE)],
        axis=1)[:, :D]


def _loss_body(lem_ref, rem_ref, out_ref, acc_ref):
    i = pl.program_id(0)
    nb = pl.num_programs(0)
    blk = lem_ref.shape[0]
    a = lem_ref[...]
    b = rem_ref[...]
    an = a / (jnp.sqrt(jnp.sum(a * a, axis=1, keepdims=True)) + 1e-9)
    bn = b / (jnp.sqrt(jnp.sum(b * b, axis=1, keepdims=True)) + 1e-9)
    sim = jax.lax.dot_general(an, bn, (((1,), (1,)), ((), ())),
                              preferred_element_type=jnp.float32)
    rows = jax.lax.broadcasted_iota(jnp.int32, sim.shape, 0) + i * blk
    cols = jax.lax.broadcasted_iota(jnp.int32, sim.shape, 1)
    diag_mask = rows == cols
    d = jnp.sum(jnp.where(diag_mask, sim, 0.0), axis=1)
    pos_c = jnp.sum(jnp.minimum(d, 0.9))
    neg_mat = jnp.where(diag_mask, 0.2, jnp.maximum(sim, 0.2))
    neg_c = jnp.sum(neg_mat)

    @pl.when(i == 0)
    def _():
        acc_ref[0, 0] = 0.0
        acc_ref[0, 1] = 0.0

    acc_ref[0, 0] += pos_c
    acc_ref[0, 1] += neg_c

    @pl.when(i == nb - 1)
    def _():
        n = jnp.float32(S)
        pos_part = acc_ref[0, 0] / n
        neg_part = acc_ref[0, 1] / (n * n)
        out_ref[0] = neg_part - 0.2 - pos_part + 0.9


def _loss_pallas(lem, rem):
    blk = 256
    grid = (S // blk,)
    return pl.pallas_call(
        _loss_body,
        grid=grid,
        in_specs=[
            pl.BlockSpec((blk, lem.shape[1]), lambda i: (i, 0)),
            pl.BlockSpec((S, rem.shape[1]), lambda i: (0, 0)),
        ],
        out_specs=pl.BlockSpec((1,), lambda i: (0,), memory_space=pltpu.SMEM),
        out_shape=jax.ShapeDtypeStruct((1,), jnp.float32),
        scratch_shapes=[pltpu.SMEM((1, 2), jnp.float32)],
    )(lem, rem)[0]


def _ragged_attn(rel_seq, head_seq, seg_ids, kernel_em, ent_feat, rel_feat,
                 Wq, Wk, Wv, Wo, n_seg):
    tok = rel_feat[rel_seq] + ent_feat[head_seq]
    q = (kernel_em @ Wq).reshape(n_seg, NH, DK)
    k = (tok @ Wk).reshape(-1, NH, DK)
    v = (tok @ Wv).reshape(-1, NH, DV)
    logits = jnp.einsum('thd,thd->th', k, q[seg_ids]) / jnp.sqrt(DK)
    m = jax.ops.segment_max(logits, seg_ids, num_segments=n_seg)
    m = jax.lax.stop_gradient(jnp.where(jnp.isfinite(m), m, 0.0))
    e = jnp.exp(logits - m[seg_ids])
    z = jax.ops.segment_sum(e, seg_ids, num_segments=n_seg)
    attn = e / (z[seg_ids] + 1e-9)
    pooled = jax.ops.segment_sum(attn[:, :, None] * v, seg_ids,
                                 num_segments=n_seg)
    return pooled.reshape(n_seg, NH * DV) @ Wo


def kernel(e_x, r_x, prim_adj, rela_adj, train_len, l_ent_ids, r_ent_ids,
           li_rel_seq, li_head_seq, li_seg_ids, lo_rel_seq, lo_head_seq,
           lo_seg_ids, ri_rel_seq, ri_head_seq, ri_seg_ids, ro_rel_seq,
           ro_head_seq, ro_seg_ids, Wrel, brel, W1, b1, W2, b2, Wq_i, Wk_i,
           Wv_i, Wo_i, Wq_o, Wk_o, Wv_o, Wo_o, Wprox, bprox):
    rel0 = r_x @ Wrel + brel
    X = jnp.concatenate([e_x, rel0], axis=0)
    N = E_CNT + R_CNT
    agg1 = _gcn_segment_sum(X, prim_adj[0], prim_adj[1])
    H1 = jax.nn.relu(agg1 @ W1 + b1)
    agg2 = _gcn_segment_sum(H1, rela_adj[0], rela_adj[1])
    out = agg2 @ W2 + b2
    gcn_ex = out[:E_CNT]
    gcn_rx = out[E_CNT:]
    Sn = l_ent_ids.shape[0]

    def prox(ent_ids, irel, ihead, iseg, orel, ohead, oseg):
        kernel_em = gcn_ex[ent_ids]
        in_em = _ragged_attn(irel, ihead, iseg, kernel_em, gcn_ex, gcn_rx,
                             Wq_i, Wk_i, Wv_i, Wo_i, Sn)
        out_em = _ragged_attn(orel, ohead, oseg, kernel_em, gcn_ex, gcn_rx,
                              Wq_o, Wk_o, Wv_o, Wo_o, Sn)
        p = jnp.concatenate([in_em, out_em], axis=1) @ Wprox + bprox
        return jnp.concatenate([kernel_em, p], axis=1)

    lem = prox(l_ent_ids, li_rel_seq, li_head_seq, li_seg_ids,
               lo_rel_seq, lo_head_seq, lo_seg_ids)
    rem = prox(r_ent_ids, ri_rel_seq, ri_head_seq, ri_seg_ids,
               ro_rel_seq, ro_head_seq, ro_seg_ids)
    loss = _loss_pallas(lem, rem)
    return (loss, lem, rem)
